# Initial kernel scaffold; baseline (speedup 1.0000x reference)
import jax
import jax.numpy as jnp
from jax.experimental import pallas as pl


def kernel(nodes, neigh_idx, features):
    raise NotImplementedError("write your pallas kernel here")


# speedup vs baseline: 3.9886x; 3.9886x over previous
import functools

import jax
import jax.numpy as jnp
from jax import lax
from jax.experimental import pallas as pl
from jax.experimental.pallas import tpu as pltpu
from jax.experimental.pallas import tpu_sc as plsc

_L = 16
_NW = 32
_S = 10
_C = 32
_G = 64
_NG = _C * _S // _G


@functools.lru_cache(maxsize=None)
def _make_sc_agg(Bp: int, D: int):
    bpw = Bp // _NW
    n_chunks = bpw // _C
    mesh = plsc.VectorSubcoreMesh(core_axis_name="c", subcore_axis_name="s")

    @functools.partial(
        pl.kernel,
        mesh=mesh,
        out_type=jax.ShapeDtypeStruct((Bp, D), jnp.float32),
        scratch_types=[
            pltpu.VMEM((_C * _S,), jnp.int32),
            pltpu.VMEM((_C * _S, D), jnp.float32),
            pltpu.VMEM((_C, D), jnp.float32),
            pltpu.SemaphoreType.DMA,
        ],
    )
    def k(features_hbm, idx_hbm, out_hbm, idx_v, rows_v, outc_v, sem):
        cid = lax.axis_index("c")
        sid = lax.axis_index("s")
        wid = sid * 2 + cid
        w_node_base = wid * bpw

        def chunk_body(ci, carry):
            nbase = w_node_base + ci * _C
            ibase = nbase * _S
            pltpu.sync_copy(idx_hbm.at[pl.ds(ibase, _C * _S)], idx_v)
            cps = [
                pltpu.async_copy(
                    features_hbm.at[idx_v.at[pl.ds(j * _G, _G)]],
                    rows_v.at[pl.ds(j * _G, _G)],
                    sem,
                )
                for j in range(_NG)
            ]
            for cp in cps:
                cp.wait()

            def node_body(n, c2):
                r0 = n * _S
                accs = [rows_v[r0, pl.ds(d * _L, _L)] for d in range(D // _L)]
                for s in range(1, _S):
                    for d in range(D // _L):
                        accs[d] = accs[d] + rows_v[r0 + s, pl.ds(d * _L, _L)]
                for d in range(D // _L):
                    outc_v[n, pl.ds(d * _L, _L)] = accs[d] * (1.0 / _S)
                return c2

            lax.fori_loop(0, _C, node_body, 0)
            pltpu.sync_copy(outc_v, out_hbm.at[pl.ds(nbase, _C)])
            return carry

        lax.fori_loop(0, n_chunks, chunk_body, 0)

    return k


def kernel(nodes, neigh_idx, features):
    B, S = neigh_idx.shape
    D = features.shape[1]
    assert S == _S and D % _L == 0
    block = _NW * _C
    Bp = ((B + block - 1) // block) * block
    idx = neigh_idx.astype(jnp.int32)
    if Bp != B:
        idx = jnp.pad(idx, ((0, Bp - B), (0, 0)))
    idx2 = idx.reshape(Bp * _S)
    out = _make_sc_agg(Bp, D)(features, idx2)
    return out[:B]


# speedup vs baseline: 5.6106x; 1.4067x over previous
import functools

import jax
import jax.numpy as jnp
from jax import lax
from jax.experimental import pallas as pl
from jax.experimental.pallas import tpu as pltpu
from jax.experimental.pallas import tpu_sc as plsc

_L = 16
_NW = 32
_S = 10
_C = 32
_G = 64
_NG = _C * _S // _G


@functools.lru_cache(maxsize=None)
def _make_sc_agg(Bp: int, D: int):
    bpw = Bp // _NW
    n_pairs = bpw // (2 * _C)
    mesh = plsc.VectorSubcoreMesh(core_axis_name="c", subcore_axis_name="s")

    @functools.partial(
        pl.kernel,
        mesh=mesh,
        out_type=jax.ShapeDtypeStruct((Bp, D), jnp.float32),
        scratch_types=[
            pltpu.VMEM((bpw * _S,), jnp.int32),
            pltpu.VMEM((2, _C * _S, D), jnp.float32),
            pltpu.VMEM((2, _C, D), jnp.float32),
            pltpu.SemaphoreType.DMA,
            pltpu.SemaphoreType.DMA,
        ],
    )
    def k(features_hbm, idx_hbm, out_hbm, idx_v, rows_v, outc_v, sem0, sem1):
        cid = lax.axis_index("c")
        sid = lax.axis_index("s")
        wid = sid * 2 + cid
        w_node_base = wid * bpw
        sems = (sem0, sem1)

        pltpu.sync_copy(idx_hbm.at[pl.ds(w_node_base * _S, bpw * _S)], idx_v)

        def fire(ci, buf):
            for j in range(_NG):
                pltpu.async_copy(
                    features_hbm.at[idx_v.at[pl.ds(ci * (_C * _S) + j * _G, _G)]],
                    rows_v.at[buf].at[pl.ds(j * _G, _G)],
                    sems[buf],
                )

        def drain(buf):
            pltpu.make_async_copy(
                features_hbm.at[pl.ds(0, _C * _S)], rows_v.at[buf], sems[buf]
            ).wait()

        def compute_store(ci, buf):
            def node_body(n, c2):
                r0 = n * _S
                accs = [rows_v[buf, r0, pl.ds(d * _L, _L)] for d in range(D // _L)]
                for s in range(1, _S):
                    for d in range(D // _L):
                        accs[d] = accs[d] + rows_v[buf, r0 + s, pl.ds(d * _L, _L)]
                for d in range(D // _L):
                    outc_v[buf, n, pl.ds(d * _L, _L)] = accs[d] * (1.0 / _S)
                return c2

            lax.fori_loop(0, _C, node_body, 0)
            nbase = w_node_base + ci * _C
            pltpu.sync_copy(outc_v.at[buf], out_hbm.at[pl.ds(nbase, _C)])

        fire(0, 0)

        def pair_body(g, carry):
            c0 = 2 * g
            fire(c0 + 1, 1)
            drain(0)
            compute_store(c0, 0)

            @pl.when(g < n_pairs - 1)
            def _():
                fire(c0 + 2, 0)

            drain(1)
            compute_store(c0 + 1, 1)
            return carry

        lax.fori_loop(0, n_pairs, pair_body, 0)

    return k


def kernel(nodes, neigh_idx, features):
    B, S = neigh_idx.shape
    D = features.shape[1]
    assert S == _S and D % _L == 0
    block = 2 * _NW * _C
    Bp = ((B + block - 1) // block) * block
    idx = neigh_idx.astype(jnp.int32)
    if Bp != B:
        idx = jnp.pad(idx, ((0, Bp - B), (0, 0)))
    idx2 = idx.reshape(Bp * _S)
    out = _make_sc_agg(Bp, D)(features, idx2)
    return out[:B]
